# Initial kernel scaffold; baseline (speedup 1.0000x reference)
import jax
import jax.numpy as jnp
from jax.experimental import pallas as pl


def kernel(nxyz, num_atoms, nbr_list, embed, gauss_offsets, conv_W_ef1, conv_b_ef1, conv_W_ef2, conv_b_ef2, conv_W_nf, conv_b_nf, conv_W_u1, conv_b_u1, conv_W_u2, conv_b_u2, W_r1, b_r1, W_r2, b_r2):
    raise NotImplementedError("write your pallas kernel here")


# speedup vs baseline: 5.1873x; 5.1873x over previous
import functools

import jax
import jax.numpy as jnp
from jax import lax
from jax.experimental import pallas as pl
from jax.experimental.pallas import tpu as pltpu
from jax.experimental.pallas import tpu_sc as plsc

_NA = 10000
_NE = 320000
_NMOL = 20
_APM = _NA // _NMOL
_NB = 128
_NG = 32
_NL = 3
_LN2 = 0.6931471805599453

_NCORES = 2
_NSUB = 16
_NW = _NCORES * _NSUB
_CH = 128
_NCHUNKS = _NE // _CH
_CPW = -(-_NCHUNKS // _NW)
_RCP = 80
_NRC = _NA // _RCP
_RCPW = -(-_NRC // _NSUB)

_FCB = 20
_FBLK = _NCHUNKS // _FCB


def _ssp(x):
    return jnp.maximum(x, 0.0) + jnp.log1p(jnp.exp(-jnp.abs(x))) - _LN2


def _filter_body(e2, offs, w1, b1, w2, b2, o0, o1, o2):
    width = offs[1, 0] - offs[0, 0]
    coeff = -0.5 / (width * width)
    offc = offs[...]
    outs = (o0, o1, o2)
    dn = (((0,), (0,)), ((), ()))
    for j in range(_FCB):
        e = jnp.sqrt(e2[0, j:j + 1, :])
        d = e - offc
        gt = jnp.exp(coeff * d * d)
        for i in range(_NL):
            h = _ssp(lax.dot_general(gt, w1[i], dn,
                                     preferred_element_type=jnp.float32) + b1[i])
            wc = jnp.dot(h, w2[i], preferred_element_type=jnp.float32) + b2[i]
            outs[i][pl.ds(j * _CH, _CH), :] = wc


def _edge_filters(e2, offs_col, w1, b1, w2, b2):
    out = jax.ShapeDtypeStruct((_NE, _NB), jnp.float32)
    full = lambda s: pl.BlockSpec(s, lambda i: tuple(0 for _ in s))
    return pl.pallas_call(
        _filter_body,
        grid=(_FBLK,),
        in_specs=[
            pl.BlockSpec((1, _FCB, _CH), lambda i: (i, 0, 0)),
            full((_NG, 1)),
            full((_NL, _NG, _NB)),
            full((_NL, _NB)),
            full((_NL, _NB, _NB)),
            full((_NL, _NB)),
        ],
        out_specs=[pl.BlockSpec((_FCB * _CH, _NB), lambda i: (i, 0))] * _NL,
        out_shape=[out] * _NL,
    )(e2.reshape(_FBLK, _FCB, _CH), offs_col, w1, b1, w2, b2)


def _node0_body(z, embed, wnf, bnf, r0, rn0):
    lanes = lax.broadcasted_iota(jnp.int32, (1, _NB), 1)
    onehot = (z[...] == lanes).astype(jnp.float32)
    r = jnp.dot(onehot, embed[...], preferred_element_type=jnp.float32)
    r0[...] = r
    rn0[...] = jnp.dot(r, wnf[...], preferred_element_type=jnp.float32) + bnf[0, :]


def _node0(z2, embed_pad, wnf, bnf2):
    out = jax.ShapeDtypeStruct((_NA, _NB), jnp.float32)
    return pl.pallas_call(_node0_body, out_shape=[out, out])(
        z2, embed_pad, wnf, bnf2)


def _update_body(parts, r, wu1, bu1, wu2, bu2, wnf, bnf, r2, rn2):
    agg = parts[0:_NA, :] + parts[_NA:2 * _NA, :]
    h = _ssp(jnp.dot(agg, wu1[...], preferred_element_type=jnp.float32) + bu1[0, :])
    dr = jnp.dot(h, wu2[...], preferred_element_type=jnp.float32) + bu2[0, :]
    rr = r[...] + dr
    r2[...] = rr
    rn2[...] = jnp.dot(rr, wnf[...], preferred_element_type=jnp.float32) + bnf[0, :]


def _node_update(parts, r, wu1, bu1, wu2, bu2, wnf, bnf):
    out = jax.ShapeDtypeStruct((_NA, _NB), jnp.float32)
    return pl.pallas_call(_update_body, out_shape=[out, out])(
        parts, r, wu1, bu1.reshape(1, _NB), wu2, bu2.reshape(1, _NB),
        wnf, bnf.reshape(1, _NB))


def _readout_body(r, w1, b1, w2, b2, out):
    h = _ssp(jnp.dot(r[0], w1[...], preferred_element_type=jnp.float32) + b1[0, :])
    ae = jnp.dot(h, w2[...], preferred_element_type=jnp.float32) + b2[0, 0]
    out[...] = jnp.sum(ae).reshape(1, 1, 1)


def _readout(r, w1, b1, w2, b2):
    nh = w1.shape[1]
    full = lambda s: pl.BlockSpec(s, lambda m: tuple(0 for _ in s))
    return pl.pallas_call(
        _readout_body,
        grid=(_NMOL,),
        in_specs=[
            pl.BlockSpec((1, _APM, _NB), lambda m: (m, 0, 0)),
            full((_NB, nh)),
            full((1, nh)),
            full((nh, 1)),
            full((1, 1)),
        ],
        out_specs=pl.BlockSpec((1, 1, 1), lambda m: (m, 0, 0)),
        out_shape=jax.ShapeDtypeStruct((_NMOL, 1, 1), jnp.float32),
    )(r.reshape(_NMOL, _APM, _NB), w1, b1.reshape(1, nh), w2, b2.reshape(1, 1))


@functools.cache
def _mesh():
    return plsc.VectorSubcoreMesh(core_axis_name="c", subcore_axis_name="s",
                                  num_cores=_NCORES, num_subcores=_NSUB)


def _sc_e2_body(src_hbm, dst_hbm, x_hbm, y_hbm, z_hbm, e2_hbm,
                xv, yv, zv, sidx, didx, e2buf):
    cid = lax.axis_index("c")
    sid = lax.axis_index("s")
    wid = sid * _NCORES + cid

    pltpu.sync_copy(x_hbm, xv)
    pltpu.sync_copy(y_hbm, yv)
    pltpu.sync_copy(z_hbm, zv)

    def chunk(k, carry):
        c = wid + k * _NW

        @pl.when(c < _NCHUNKS)
        def _():
            base = c * _CH
            pltpu.sync_copy(src_hbm.at[pl.ds(base, _CH)], sidx)
            pltpu.sync_copy(dst_hbm.at[pl.ds(base, _CH)], didx)
            for g in range(_CH // 16):
                s = pl.ds(g * 16, 16)
                si = sidx[s]
                di = didx[s]
                dx = plsc.load_gather(xv, [si]) - plsc.load_gather(xv, [di])
                dy = plsc.load_gather(yv, [si]) - plsc.load_gather(yv, [di])
                dz = plsc.load_gather(zv, [si]) - plsc.load_gather(zv, [di])
                e2buf[s] = dx * dx + dy * dy + dz * dz
            pltpu.sync_copy(e2buf, e2_hbm.at[pl.ds(base, _CH)])

        return carry

    lax.fori_loop(0, _CPW, chunk, 0)


@functools.cache
def _sc_e2():
    return pl.kernel(
        _sc_e2_body,
        out_type=jax.ShapeDtypeStruct((_NE,), jnp.float32),
        mesh=_mesh(),
        scratch_types=[
            pltpu.VMEM((_NA,), jnp.float32),
            pltpu.VMEM((_NA,), jnp.float32),
            pltpu.VMEM((_NA,), jnp.float32),
            pltpu.VMEM((_CH,), jnp.int32),
            pltpu.VMEM((_CH,), jnp.int32),
            pltpu.VMEM((_CH,), jnp.float32),
        ],
        compiler_params=pltpu.CompilerParams(needs_layout_passes=False),
    )


def _sc_msg_body(src_hbm, dst_hbm, rn_hbm, w_hbm, out_hbm,
                 sidx, didx, abuf, bbuf, wbuf, agg,
                 sem_a, sem_b, sem_w):
    cid = lax.axis_index("c")
    sid = lax.axis_index("s")
    wid = sid * _NCORES + cid

    def zrow(i, carry):
        for j in range(_NB // 16):
            abuf[i, pl.ds(j * 16, 16)] = jnp.zeros((16,), jnp.float32)
        return carry

    lax.fori_loop(0, _CH, zrow, 0)
    for k in range(_RCPW):
        c = sid + k * _NSUB

        @pl.when(c < _NRC)
        def _():
            pltpu.sync_copy(abuf.at[pl.ds(0, _RCP)],
                            agg.at[pl.ds(c * _RCP, _RCP)])

    plsc.subcore_barrier()

    def chunk(k, carry):
        c = wid + k * _NW

        @pl.when(c < _NCHUNKS)
        def _():
            base = c * _CH
            pltpu.sync_copy(src_hbm.at[pl.ds(base, _CH)], sidx)
            pltpu.sync_copy(dst_hbm.at[pl.ds(base, _CH)], didx)
            ca = pltpu.async_copy(rn_hbm.at[sidx], abuf, sem_a)
            cb = pltpu.async_copy(rn_hbm.at[didx], bbuf, sem_b)
            cw = pltpu.async_copy(w_hbm.at[pl.ds(base, _CH)], wbuf, sem_w)
            ca.wait()
            cb.wait()
            cw.wait()

            def mrow(i, cc):
                for j in range(_NB // 16):
                    s = pl.ds(j * 16, 16)
                    w = wbuf[i, s]
                    abuf[i, s] = abuf[i, s] * w
                    bbuf[i, s] = bbuf[i, s] * w
                return cc

            lax.fori_loop(0, _CH, mrow, 0)
            pltpu.sync_copy(abuf, agg.at[didx], add=True)
            pltpu.sync_copy(bbuf, agg.at[sidx], add=True)

        return carry

    lax.fori_loop(0, _CPW, chunk, 0)
    plsc.subcore_barrier()

    for k in range(_RCPW):
        c = sid + k * _NSUB

        @pl.when(c < _NRC)
        def _():
            off = c * _RCP
            pltpu.sync_copy(agg.at[pl.ds(off, _RCP)],
                            out_hbm.at[pl.ds(cid * _NA + off, _RCP)])


@functools.cache
def _sc_message():
    return pl.kernel(
        _sc_msg_body,
        out_type=jax.ShapeDtypeStruct((2 * _NA, _NB), jnp.float32),
        mesh=_mesh(),
        scratch_types=[
            pltpu.VMEM((_CH,), jnp.int32),
            pltpu.VMEM((_CH,), jnp.int32),
            pltpu.VMEM((_CH, _NB), jnp.float32),
            pltpu.VMEM((_CH, _NB), jnp.float32),
            pltpu.VMEM((_CH, _NB), jnp.float32),
            pltpu.VMEM_SHARED((_NA, _NB), jnp.float32),
            pltpu.SemaphoreType.DMA,
            pltpu.SemaphoreType.DMA,
            pltpu.SemaphoreType.DMA,
        ],
    )


def kernel(nxyz, num_atoms, nbr_list, embed, gauss_offsets,
           conv_W_ef1, conv_b_ef1, conv_W_ef2, conv_b_ef2,
           conv_W_nf, conv_b_nf, conv_W_u1, conv_b_u1,
           conv_W_u2, conv_b_u2, W_r1, b_r1, W_r2, b_r2):
    xyz = nxyz[:, 1:4]
    z = nxyz[:, 0].astype(jnp.int32)
    src = nbr_list[:, 0].astype(jnp.int32)
    dst = nbr_list[:, 1].astype(jnp.int32)

    embed_pad = jnp.pad(embed, ((0, _NB - embed.shape[0]), (0, 0)))

    e2 = _sc_e2()(src, dst, xyz[:, 0], xyz[:, 1], xyz[:, 2])

    w_all = _edge_filters(e2, gauss_offsets.reshape(_NG, 1),
                          conv_W_ef1, conv_b_ef1, conv_W_ef2, conv_b_ef2)

    r, rn = _node0(z.reshape(_NA, 1), embed_pad, conv_W_nf[0],
                   conv_b_nf[0].reshape(1, _NB))

    for i in range(_NL):
        parts = _sc_message()(src, dst, rn, w_all[i])
        nxt = (i + 1) % _NL
        r, rn = _node_update(parts, r, conv_W_u1[i], conv_b_u1[i],
                             conv_W_u2[i], conv_b_u2[i],
                             conv_W_nf[nxt], conv_b_nf[nxt])

    energy = _readout(r, W_r1, b_r1, W_r2, b_r2)
    return energy.reshape(_NMOL)


# speedup vs baseline: 5.3801x; 1.0372x over previous
import functools

import jax
import jax.numpy as jnp
from jax import lax
from jax.experimental import pallas as pl
from jax.experimental.pallas import tpu as pltpu
from jax.experimental.pallas import tpu_sc as plsc

_NA = 10000
_NE = 320000
_NMOL = 20
_APM = _NA // _NMOL
_NB = 128
_NG = 32
_NL = 3
_LN2 = 0.6931471805599453

_NCORES = 2
_NSUB = 16
_NW = _NCORES * _NSUB
_CH = 128
_NCHUNKS = _NE // _CH
_CPW = -(-_NCHUNKS // _NW)
_RCP = 80
_NRC = _NA // _RCP
_RCPW = -(-_NRC // _NSUB)

_FCB = 20
_FBLK = _NCHUNKS // _FCB


def _ssp(x):
    return jnp.maximum(x, 0.0) + jnp.log1p(jnp.exp(-jnp.abs(x))) - _LN2


def _filter_body(e2, offs, w1, b1, w2, b2, o0, o1, o2):
    width = offs[1, 0] - offs[0, 0]
    coeff = -0.5 / (width * width)
    offc = offs[...]
    outs = (o0, o1, o2)
    dn = (((0,), (0,)), ((), ()))
    for j in range(_FCB):
        e = jnp.sqrt(e2[0, j:j + 1, :])
        d = e - offc
        gt = jnp.exp(coeff * d * d)
        for i in range(_NL):
            h = _ssp(lax.dot_general(gt, w1[i], dn,
                                     preferred_element_type=jnp.float32) + b1[i])
            wc = jnp.dot(h, w2[i], preferred_element_type=jnp.float32) + b2[i]
            outs[i][pl.ds(j * _CH, _CH), :] = wc


def _edge_filters(e2, offs_col, w1, b1, w2, b2):
    out = jax.ShapeDtypeStruct((_NE, _NB), jnp.float32)
    full = lambda s: pl.BlockSpec(s, lambda i: tuple(0 for _ in s))
    return pl.pallas_call(
        _filter_body,
        grid=(_FBLK,),
        in_specs=[
            pl.BlockSpec((1, _FCB, _CH), lambda i: (i, 0, 0)),
            full((_NG, 1)),
            full((_NL, _NG, _NB)),
            full((_NL, _NB)),
            full((_NL, _NB, _NB)),
            full((_NL, _NB)),
        ],
        out_specs=[pl.BlockSpec((_FCB * _CH, _NB), lambda i: (i, 0))] * _NL,
        out_shape=[out] * _NL,
    )(e2.reshape(_FBLK, _FCB, _CH), offs_col, w1, b1, w2, b2)


def _node0_body(z, embed, wnf, bnf, r0, rn0):
    lanes = lax.broadcasted_iota(jnp.int32, (1, _NB), 1)
    onehot = (z[...] == lanes).astype(jnp.float32)
    r = jnp.dot(onehot, embed[...], preferred_element_type=jnp.float32)
    r0[...] = r
    rn0[...] = jnp.dot(r, wnf[...], preferred_element_type=jnp.float32) + bnf[0, :]


def _node0(z2, embed_pad, wnf, bnf2):
    out = jax.ShapeDtypeStruct((_NA, _NB), jnp.float32)
    return pl.pallas_call(_node0_body, out_shape=[out, out])(
        z2, embed_pad, wnf, bnf2)


def _update_body(parts, r, wu1, bu1, wu2, bu2, wnf, bnf, r2, rn2):
    agg = parts[0:_NA, :] + parts[_NA:2 * _NA, :]
    h = _ssp(jnp.dot(agg, wu1[...], preferred_element_type=jnp.float32) + bu1[0, :])
    dr = jnp.dot(h, wu2[...], preferred_element_type=jnp.float32) + bu2[0, :]
    rr = r[...] + dr
    r2[...] = rr
    rn2[...] = jnp.dot(rr, wnf[...], preferred_element_type=jnp.float32) + bnf[0, :]


def _node_update(parts, r, wu1, bu1, wu2, bu2, wnf, bnf):
    out = jax.ShapeDtypeStruct((_NA, _NB), jnp.float32)
    return pl.pallas_call(_update_body, out_shape=[out, out])(
        parts, r, wu1, bu1.reshape(1, _NB), wu2, bu2.reshape(1, _NB),
        wnf, bnf.reshape(1, _NB))


def _readout_body(r, w1, b1, w2, b2, out):
    h = _ssp(jnp.dot(r[0], w1[...], preferred_element_type=jnp.float32) + b1[0, :])
    ae = jnp.dot(h, w2[...], preferred_element_type=jnp.float32) + b2[0, 0]
    out[...] = jnp.sum(ae).reshape(1, 1, 1)


def _readout(r, w1, b1, w2, b2):
    nh = w1.shape[1]
    full = lambda s: pl.BlockSpec(s, lambda m: tuple(0 for _ in s))
    return pl.pallas_call(
        _readout_body,
        grid=(_NMOL,),
        in_specs=[
            pl.BlockSpec((1, _APM, _NB), lambda m: (m, 0, 0)),
            full((_NB, nh)),
            full((1, nh)),
            full((nh, 1)),
            full((1, 1)),
        ],
        out_specs=pl.BlockSpec((1, 1, 1), lambda m: (m, 0, 0)),
        out_shape=jax.ShapeDtypeStruct((_NMOL, 1, 1), jnp.float32),
    )(r.reshape(_NMOL, _APM, _NB), w1, b1.reshape(1, nh), w2, b2.reshape(1, 1))


@functools.cache
def _mesh():
    return plsc.VectorSubcoreMesh(core_axis_name="c", subcore_axis_name="s",
                                  num_cores=_NCORES, num_subcores=_NSUB)


def _sc_e2_body(src_hbm, dst_hbm, x_hbm, y_hbm, z_hbm, e2_hbm,
                xv, yv, zv, sidx, didx, e2buf):
    cid = lax.axis_index("c")
    sid = lax.axis_index("s")
    wid = sid * _NCORES + cid

    epw = _NE // _NW
    pltpu.sync_copy(x_hbm, xv)
    pltpu.sync_copy(y_hbm, yv)
    pltpu.sync_copy(z_hbm, zv)
    pltpu.sync_copy(src_hbm.at[pl.ds(wid * epw, epw)], sidx)
    pltpu.sync_copy(dst_hbm.at[pl.ds(wid * epw, epw)], didx)

    nbuf = 2000
    for t in range(epw // nbuf):
        def grp(g, carry):
            s = pl.ds(t * nbuf + g * 16, 16)
            si = sidx[s]
            di = didx[s]
            dx = plsc.load_gather(xv, [si]) - plsc.load_gather(xv, [di])
            dy = plsc.load_gather(yv, [si]) - plsc.load_gather(yv, [di])
            dz = plsc.load_gather(zv, [si]) - plsc.load_gather(zv, [di])
            e2buf[pl.ds(g * 16, 16)] = dx * dx + dy * dy + dz * dz
            return carry

        lax.fori_loop(0, nbuf // 16, grp, 0)
        pltpu.sync_copy(e2buf, e2_hbm.at[pl.ds(wid * epw + t * nbuf, nbuf)])


@functools.cache
def _sc_e2():
    return pl.kernel(
        _sc_e2_body,
        out_type=jax.ShapeDtypeStruct((_NE,), jnp.float32),
        mesh=_mesh(),
        scratch_types=[
            pltpu.VMEM((_NA,), jnp.float32),
            pltpu.VMEM((_NA,), jnp.float32),
            pltpu.VMEM((_NA,), jnp.float32),
            pltpu.VMEM((_NE // _NW,), jnp.int32),
            pltpu.VMEM((_NE // _NW,), jnp.int32),
            pltpu.VMEM((2000,), jnp.float32),
        ],
        compiler_params=pltpu.CompilerParams(needs_layout_passes=False),
    )


def _sc_msg_body(src_hbm, dst_hbm, rn_hbm, w_hbm, out_hbm,
                 sidx, didx, abuf, bbuf, wbuf, agg,
                 sem_a, sem_b, sem_w):
    cid = lax.axis_index("c")
    sid = lax.axis_index("s")
    wid = sid * _NCORES + cid

    def zrow(i, carry):
        for j in range(_NB // 16):
            abuf[i, pl.ds(j * 16, 16)] = jnp.zeros((16,), jnp.float32)
        return carry

    lax.fori_loop(0, _CH, zrow, 0)
    for k in range(_RCPW):
        c = sid + k * _NSUB

        @pl.when(c < _NRC)
        def _():
            pltpu.sync_copy(abuf.at[pl.ds(0, _RCP)],
                            agg.at[pl.ds(c * _RCP, _RCP)])

    plsc.subcore_barrier()

    def chunk(k, carry):
        c = wid + k * _NW

        @pl.when(c < _NCHUNKS)
        def _():
            base = c * _CH
            pltpu.sync_copy(src_hbm.at[pl.ds(base, _CH)], sidx)
            pltpu.sync_copy(dst_hbm.at[pl.ds(base, _CH)], didx)
            ca = pltpu.async_copy(rn_hbm.at[sidx], abuf, sem_a)
            cb = pltpu.async_copy(rn_hbm.at[didx], bbuf, sem_b)
            cw = pltpu.async_copy(w_hbm.at[pl.ds(base, _CH)], wbuf, sem_w)
            ca.wait()
            cb.wait()
            cw.wait()

            def mrow(i, cc):
                for j in range(_NB // 16):
                    s = pl.ds(j * 16, 16)
                    w = wbuf[i, s]
                    abuf[i, s] = abuf[i, s] * w
                    bbuf[i, s] = bbuf[i, s] * w
                return cc

            lax.fori_loop(0, _CH, mrow, 0)
            pltpu.sync_copy(abuf, agg.at[didx], add=True)
            pltpu.sync_copy(bbuf, agg.at[sidx], add=True)

        return carry

    lax.fori_loop(0, _CPW, chunk, 0)
    plsc.subcore_barrier()

    for k in range(_RCPW):
        c = sid + k * _NSUB

        @pl.when(c < _NRC)
        def _():
            off = c * _RCP
            pltpu.sync_copy(agg.at[pl.ds(off, _RCP)],
                            out_hbm.at[pl.ds(cid * _NA + off, _RCP)])


@functools.cache
def _sc_message():
    return pl.kernel(
        _sc_msg_body,
        out_type=jax.ShapeDtypeStruct((2 * _NA, _NB), jnp.float32),
        mesh=_mesh(),
        scratch_types=[
            pltpu.VMEM((_CH,), jnp.int32),
            pltpu.VMEM((_CH,), jnp.int32),
            pltpu.VMEM((_CH, _NB), jnp.float32),
            pltpu.VMEM((_CH, _NB), jnp.float32),
            pltpu.VMEM((_CH, _NB), jnp.float32),
            pltpu.VMEM_SHARED((_NA, _NB), jnp.float32),
            pltpu.SemaphoreType.DMA,
            pltpu.SemaphoreType.DMA,
            pltpu.SemaphoreType.DMA,
        ],
    )


def kernel(nxyz, num_atoms, nbr_list, embed, gauss_offsets,
           conv_W_ef1, conv_b_ef1, conv_W_ef2, conv_b_ef2,
           conv_W_nf, conv_b_nf, conv_W_u1, conv_b_u1,
           conv_W_u2, conv_b_u2, W_r1, b_r1, W_r2, b_r2):
    xyz = nxyz[:, 1:4]
    z = nxyz[:, 0].astype(jnp.int32)
    src = nbr_list[:, 0].astype(jnp.int32)
    dst = nbr_list[:, 1].astype(jnp.int32)

    embed_pad = jnp.pad(embed, ((0, _NB - embed.shape[0]), (0, 0)))

    e2 = _sc_e2()(src, dst, xyz[:, 0], xyz[:, 1], xyz[:, 2])

    w_all = _edge_filters(e2, gauss_offsets.reshape(_NG, 1),
                          conv_W_ef1, conv_b_ef1, conv_W_ef2, conv_b_ef2)

    r, rn = _node0(z.reshape(_NA, 1), embed_pad, conv_W_nf[0],
                   conv_b_nf[0].reshape(1, _NB))

    for i in range(_NL):
        parts = _sc_message()(src, dst, rn, w_all[i])
        nxt = (i + 1) % _NL
        r, rn = _node_update(parts, r, conv_W_u1[i], conv_b_u1[i],
                             conv_W_u2[i], conv_b_u2[i],
                             conv_W_nf[nxt], conv_b_nf[nxt])

    energy = _readout(r, W_r1, b_r1, W_r2, b_r2)
    return energy.reshape(_NMOL)
